# baseline (speedup 1.0000x reference)
import jax
import jax.numpy as jnp
from jax.experimental import pallas as pl
from jax.experimental.pallas import tpu as pltpu


def _expert0_matmul_kernel(x_ref, wt_ref, o_ref):
    o_ref[...] = jax.lax.dot_general(
        x_ref[...],
        wt_ref[...],
        dimension_numbers=(((1,), (0,)), ((), ())),
        preferred_element_type=jnp.float32,
    )


def kernel(inputs, router_w, expert_ws):
    del router_w
    x = inputs.astype(jnp.bfloat16)
    wt = expert_ws[0].T.astype(jnp.bfloat16)
    m, k = inputs.shape
    n = wt.shape[1]
    bm = 512
    return pl.pallas_call(
        _expert0_matmul_kernel,
        grid=(m // bm,),
        in_specs=[
            pl.BlockSpec((bm, k), lambda i: (i, 0)),
            pl.BlockSpec((k, n), lambda i: (0, 0)),
        ],
        out_specs=pl.BlockSpec((bm, n), lambda i: (i, 0)),
        out_shape=jax.ShapeDtypeStruct((m, n), inputs.dtype),
        compiler_params=pltpu.CompilerParams(
            dimension_semantics=("parallel",),
        ),
    )(x, wt)


# speedup vs baseline: 1.3104x; 1.3104x over previous
import jax
import jax.numpy as jnp
from jax.experimental import pallas as pl
from jax.experimental.pallas import tpu as pltpu


def _expert0_matmul_kernel(x_ref, wt_ref, o_ref):
    o_ref[...] = jax.lax.dot_general(
        x_ref[...],
        wt_ref[...],
        dimension_numbers=(((1,), (0,)), ((), ())),
        precision=jax.lax.Precision.DEFAULT,
        preferred_element_type=jnp.float32,
    )


def kernel(inputs, router_w, expert_ws):
    del router_w
    wt = expert_ws[0].T
    m, k = inputs.shape
    n = wt.shape[1]
    bm = 512
    return pl.pallas_call(
        _expert0_matmul_kernel,
        grid=(m // bm,),
        in_specs=[
            pl.BlockSpec((bm, k), lambda i: (i, 0)),
            pl.BlockSpec((k, n), lambda i: (0, 0)),
        ],
        out_specs=pl.BlockSpec((bm, n), lambda i: (i, 0)),
        out_shape=jax.ShapeDtypeStruct((m, n), inputs.dtype),
        compiler_params=pltpu.CompilerParams(
            dimension_semantics=("parallel",),
        ),
    )(inputs, wt)


# speedup vs baseline: 1.4671x; 1.1196x over previous
import jax
import jax.numpy as jnp
from jax.experimental import pallas as pl
from jax.experimental.pallas import tpu as pltpu


def _expert0_matmul_kernel(x_ref, wt_ref, o_ref):
    o_ref[...] = jax.lax.dot_general(
        x_ref[...],
        wt_ref[...],
        dimension_numbers=(((1,), (0,)), ((), ())),
        preferred_element_type=jnp.float32,
    )


def kernel(inputs, router_w, expert_ws):
    del router_w
    wt = expert_ws[0].T
    m, k = inputs.shape
    n = wt.shape[1]
    bm = 1024
    return pl.pallas_call(
        _expert0_matmul_kernel,
        grid=(m // bm,),
        in_specs=[
            pl.BlockSpec((bm, k), lambda i: (i, 0)),
            pl.BlockSpec((k, n), lambda i: (0, 0)),
        ],
        out_specs=pl.BlockSpec((bm, n), lambda i: (i, 0)),
        out_shape=jax.ShapeDtypeStruct((m, n), inputs.dtype),
        compiler_params=pltpu.CompilerParams(
            dimension_semantics=("parallel",),
        ),
    )(inputs, wt)


# speedup vs baseline: 1.4859x; 1.0128x over previous
import jax
import jax.numpy as jnp
from jax.experimental import pallas as pl
from jax.experimental.pallas import tpu as pltpu


def _expert0_matmul_kernel(x_ref, wt_ref, o_ref):
    o_ref[...] = jax.lax.dot_general(
        x_ref[...],
        wt_ref[...],
        dimension_numbers=(((1,), (0,)), ((), ())),
        preferred_element_type=jnp.float32,
    )


def kernel(inputs, router_w, expert_ws):
    del router_w
    wt = expert_ws[0].T
    m, k = inputs.shape
    n = wt.shape[1]
    bm = 2048
    return pl.pallas_call(
        _expert0_matmul_kernel,
        grid=(m // bm,),
        in_specs=[
            pl.BlockSpec((bm, k), lambda i: (i, 0)),
            pl.BlockSpec((k, n), lambda i: (0, 0)),
        ],
        out_specs=pl.BlockSpec((bm, n), lambda i: (i, 0)),
        out_shape=jax.ShapeDtypeStruct((m, n), inputs.dtype),
        compiler_params=pltpu.CompilerParams(
            dimension_semantics=("parallel",),
        ),
    )(inputs, wt)


# speedup vs baseline: 1.5497x; 1.0430x over previous
import jax
import jax.numpy as jnp
from jax.experimental import pallas as pl
from jax.experimental.pallas import tpu as pltpu


def _expert0_matmul_kernel(x_ref, w_ref, o_ref):
    o_ref[...] = jax.lax.dot_general(
        x_ref[...],
        w_ref[...],
        dimension_numbers=(((1,), (1,)), ((), ())),
        preferred_element_type=jnp.float32,
    )


def kernel(inputs, router_w, expert_ws):
    del router_w
    w0 = expert_ws[0]
    m, k = inputs.shape
    n = w0.shape[0]
    bm = 2048
    return pl.pallas_call(
        _expert0_matmul_kernel,
        grid=(m // bm,),
        in_specs=[
            pl.BlockSpec((bm, k), lambda i: (i, 0)),
            pl.BlockSpec((n, k), lambda i: (0, 0)),
        ],
        out_specs=pl.BlockSpec((bm, n), lambda i: (i, 0)),
        out_shape=jax.ShapeDtypeStruct((m, n), inputs.dtype),
        compiler_params=pltpu.CompilerParams(
            dimension_semantics=("parallel",),
        ),
    )(inputs, w0)
